# baseline (speedup 1.0000x reference)
import functools

import numpy as np
import jax
import jax.numpy as jnp
from jax import lax
from jax.experimental import pallas as pl
from jax.experimental.pallas import tpu as pltpu
from jax.experimental.pallas import tpu_sc as plsc

BEV = 200
NQ = BEV * BEV
E = 128
NH = 8
HD = 16
NPAIR = NQ * NH

NSPLIT = 2
NQH = NQ // NSPLIT
NPAIRH = NQH * NH

BLK = 1000
VBLK = 2000

NC, NS = 2, 16
NW = NC * NS
PW = NPAIRH // NW
KP = 200
NCH = PW // KP
NJ = KP * 16 // 128
TILE_IDXROWS = PW * 16 // 128

_PREC = jax.lax.Precision.DEFAULT

_LANE = np.arange(128)
_LH = _LANE // 16
_LC = (_LANE % 16) // 4
_LP = _LANE % 4
_SRCX = _LH * 8 + _LP * 2 + 0
_SRCY = _LH * 8 + _LP * 2 + 1
_SRCA = _LH * 4 + _LP
_G16 = np.kron(np.eye(NH, dtype=np.float32), np.ones((16, 16), np.float32))


def _dot(a, b):
    return jax.lax.dot_general(a, b, (((1,), (0,)), ((), ())),
                               precision=_PREC,
                               preferred_element_type=jnp.float32)


def _val_body(q_ref, wv_ref, bv_ref, val_ref):
    val_ref[...] = _dot(q_ref[...], wv_ref[...]) + bv_ref[...]


def _pre_body(qbase, q_ref, wox_ref, box_ref, woy_ref, boy_ref,
              wat_ref, bat_ref, g16_ref, idx_ref, wts_ref):
    i = pl.program_id(0)
    q = q_ref[...]
    ox = _dot(q, wox_ref[...]) + box_ref[...]
    oy = _dot(q, woy_ref[...]) + boy_ref[...]
    lg = _dot(q, wat_ref[...]) + bat_ref[...]
    m = jnp.max(lg, axis=1, keepdims=True)
    e = jnp.exp(lg - m)
    aw = 4.0 * e / _dot(e, g16_ref[...])

    lane = lax.broadcasted_iota(jnp.int32, (BLK, E), 1)
    xsel = ((lane % 16) // 4 % 2).astype(jnp.float32)
    ysel = ((lane % 16) // 8).astype(jnp.float32)
    hf = (lane // 16).astype(jnp.float32)
    qi = qbase + i * BLK + lax.broadcasted_iota(jnp.int32, (BLK, E), 0)
    refx = (qi % BEV).astype(jnp.float32) * (1.0 / 199.0)
    refy = (qi // BEV).astype(jnp.float32) * (1.0 / 199.0)

    x = refx * 200.0 - 0.5 + ox
    y = refy * 200.0 - 0.5 + oy
    x0 = jnp.floor(x)
    y0 = jnp.floor(y)
    fx = x - x0
    fy = y - y0
    xf = x0 + xsel
    yf = y0 + ysel
    wx = (1.0 - xsel) + (2.0 * xsel - 1.0) * fx
    wy = (1.0 - ysel) + (2.0 * ysel - 1.0) * fy
    v = ((xf >= 0.0) & (xf <= 199.0) & (yf >= 0.0) & (yf <= 199.0))
    xc = jnp.clip(xf, 0.0, 199.0)
    yc = jnp.clip(yf, 0.0, 199.0)

    idx_ref[...] = ((yc * 200.0 + xc) * 8.0 + hf).astype(jnp.int32)
    wts_ref[...] = aw * wx * wy * v.astype(jnp.float32)


def _post_body(s_ref, wo_ref, bo_ref, q_ref, out_ref):
    out_ref[...] = (_dot(s_ref[...], wo_ref[...]) + bo_ref[...]
                    + 2.0 * q_ref[...])


def _full(shape):
    return pl.BlockSpec(shape, lambda i: (0, 0))


_val_call = pl.pallas_call(
    _val_body,
    grid=(NQ // VBLK,),
    in_specs=[
        pl.BlockSpec((VBLK, E), lambda i: (i, 0)),
        _full((E, E)), _full((1, E)),
    ],
    out_specs=pl.BlockSpec((VBLK, E), lambda i: (i, 0)),
    out_shape=jax.ShapeDtypeStruct((NQ, E), jnp.float32),
)


def _make_pre(qbase):
    return pl.pallas_call(
        functools.partial(_pre_body, qbase),
        grid=(NQH // BLK,),
        in_specs=[
            pl.BlockSpec((BLK, E), lambda i: (i, 0)),
            _full((E, E)), _full((1, E)),
            _full((E, E)), _full((1, E)),
            _full((E, E)), _full((1, E)),
            _full((E, E)),
        ],
        out_specs=[
            pl.BlockSpec((BLK, E), lambda i: (i, 0)),
            pl.BlockSpec((BLK, E), lambda i: (i, 0)),
        ],
        out_shape=[
            jax.ShapeDtypeStruct((NQH, E), jnp.int32),
            jax.ShapeDtypeStruct((NQH, E), jnp.float32),
        ],
    )


_pre_calls = [_make_pre(h * NQH) for h in range(NSPLIT)]

_post_call = pl.pallas_call(
    _post_body,
    grid=(NQH // BLK,),
    in_specs=[
        pl.BlockSpec((BLK, E), lambda i: (i, 0)),
        _full((E, E)), _full((1, E)),
        pl.BlockSpec((BLK, E), lambda i: (i, 0)),
    ],
    out_specs=pl.BlockSpec((BLK, E), lambda i: (i, 0)),
    out_shape=jax.ShapeDtypeStruct((NQH, E), jnp.float32),
)


def _sc_body(tab_ref, idx_ref, w_ref, out_ref,
             idx_v, w_v, rows_v, out_v, sidx, sgat, sout):
    wid = lax.axis_index("s") * NC + lax.axis_index("c")
    pairbase = wid * PW
    idxbase = wid * TILE_IDXROWS

    def fire_idx(g):
        r = lax.rem(g, 3)
        pltpu.async_copy(idx_ref.at[pl.ds(idxbase + g * NJ, NJ)],
                         idx_v.at[pl.ds(r * NJ, NJ)], sidx)
        pltpu.async_copy(w_ref.at[pl.ds(pairbase + g * KP, KP)],
                         w_v.at[pl.ds(r * KP, KP)], sidx)

    def drain_idx():
        pltpu.make_async_copy(idx_ref.at[pl.ds(0, NJ)],
                              idx_v.at[pl.ds(0, NJ)], sidx).wait()
        pltpu.make_async_copy(w_ref.at[pl.ds(0, KP)],
                              w_v.at[pl.ds(0, KP)], sidx).wait()

    def fire_gat(g, b):
        r = lax.rem(g, 3)
        for j in range(NJ):
            pltpu.async_copy(tab_ref.at[idx_v.at[r * NJ + j]],
                             rows_v.at[pl.ds(b * KP * 16 + j * 128, 128)],
                             sgat)

    def drain_gat():
        pltpu.make_async_copy(tab_ref.at[pl.ds(0, KP * 16)],
                              rows_v.at[pl.ds(0, KP * 16)], sgat).wait()

    def drain_out():
        pltpu.make_async_copy(out_v.at[pl.ds(0, KP)],
                              out_ref.at[pl.ds(0, KP)], sout).wait()

    fire_idx(0)
    drain_idx()
    fire_gat(0, 0)
    fire_idx(1)

    def loop(g, carry):
        b = lax.rem(g, 2)

        @pl.when(g + 1 < NCH)
        def _():
            drain_idx()
            fire_gat(g + 1, 1 - b)

        @pl.when(g + 2 < NCH)
        def _():
            fire_idx(g + 2)

        @pl.when(g >= 2)
        def _():
            drain_out()

        drain_gat()

        rbase = b * KP * 16
        wbase = lax.rem(g, 3) * KP
        obase = b * KP

        def pair2(i2, c2):
            i = 2 * i2
            base = rbase + i * 16
            wv0 = w_v[wbase + i]
            wv1 = w_v[wbase + i + 1]
            a0 = rows_v[base + 0] * wv0[0]
            a1 = rows_v[base + 1] * wv0[1]
            a2 = rows_v[base + 2] * wv0[2]
            a3 = rows_v[base + 3] * wv0[3]
            b0 = rows_v[base + 16] * wv1[0]
            b1 = rows_v[base + 17] * wv1[1]
            b2 = rows_v[base + 18] * wv1[2]
            b3 = rows_v[base + 19] * wv1[3]
            for l in range(4, 16, 4):
                a0 = a0 + rows_v[base + l] * wv0[l]
                a1 = a1 + rows_v[base + l + 1] * wv0[l + 1]
                a2 = a2 + rows_v[base + l + 2] * wv0[l + 2]
                a3 = a3 + rows_v[base + l + 3] * wv0[l + 3]
                b0 = b0 + rows_v[base + 16 + l] * wv1[l]
                b1 = b1 + rows_v[base + 17 + l] * wv1[l + 1]
                b2 = b2 + rows_v[base + 18 + l] * wv1[l + 2]
                b3 = b3 + rows_v[base + 19 + l] * wv1[l + 3]
            out_v[obase + i] = (a0 + a1) + (a2 + a3)
            out_v[obase + i + 1] = (b0 + b1) + (b2 + b3)
            return c2

        lax.fori_loop(0, KP // 2, pair2, 0)
        pltpu.async_copy(out_v.at[pl.ds(obase, KP)],
                         out_ref.at[pl.ds(pairbase + g * KP, KP)], sout)
        return carry

    lax.fori_loop(0, NCH, loop, 0)
    drain_out()
    drain_out()


@functools.cache
def _sc_call():
    return functools.partial(
        pl.kernel,
        out_type=jax.ShapeDtypeStruct((NPAIRH, HD), jnp.float32),
        mesh=plsc.VectorSubcoreMesh(core_axis_name="c", subcore_axis_name="s"),
        compiler_params=pltpu.CompilerParams(use_tc_tiling_on_sc=False),
        scratch_types=[
            pltpu.VMEM((3 * NJ, 128), jnp.int32),
            pltpu.VMEM((3 * KP, HD), jnp.float32),
            pltpu.VMEM((2 * KP * 16, HD), jnp.float32),
            pltpu.VMEM((2 * KP, HD), jnp.float32),
            pltpu.SemaphoreType.DMA,
            pltpu.SemaphoreType.DMA,
            pltpu.SemaphoreType.DMA,
        ],
    )(_sc_body)


def kernel(query, Wv, bv, Woff, boff, Wattn, battn, Wout, bout):
    q2 = query[0]
    wox = Woff[:, _SRCX]
    woy = Woff[:, _SRCY]
    box = boff[_SRCX][None, :]
    boy = boff[_SRCY][None, :]
    wat = Wattn[:, _SRCA]
    bat = battn[_SRCA][None, :]
    g16 = jnp.asarray(_G16)

    value = _val_call(q2, Wv, bv[None, :])
    tab = value.reshape(NPAIR, HD)

    outs = []
    for h in range(NSPLIT):
        qh = lax.slice_in_dim(q2, h * NQH, (h + 1) * NQH, axis=0)
        widx, wwts = _pre_calls[h](qh, wox, box, woy, boy, wat, bat, g16)
        srows = _sc_call()(tab, widx, wwts.reshape(NPAIRH, HD))
        outs.append(_post_call(srows.reshape(NQH, E), Wout,
                               bout[None, :], qh))
    return jnp.concatenate(outs, axis=0)[None]


# speedup vs baseline: 1.1148x; 1.1148x over previous
import functools

import numpy as np
import jax
import jax.numpy as jnp
from jax import lax
from jax.experimental import pallas as pl
from jax.experimental.pallas import tpu as pltpu
from jax.experimental.pallas import tpu_sc as plsc

BEV = 200
NQ = BEV * BEV
E = 128
NH = 8
HD = 16
NPAIR = NQ * NH

BLK = 1000
VBLK = 2000
SPLITS = (12000, 28000)

NC, NS = 2, 16
NW = NC * NS
KP = 200
NJ = KP * 16 // 128

_PREC = jax.lax.Precision.DEFAULT
_EXACT = jax.lax.Precision.HIGHEST

_LANE = np.arange(128)
_LH = _LANE // 16
_LP = _LANE % 4
_SRCX = _LH * 8 + _LP * 2 + 0
_SRCY = _LH * 8 + _LP * 2 + 1
_SRCA = _LH * 4 + _LP
_SX = (np.arange(64)[:, None] == _SRCX[None, :]).astype(np.float32)
_SY = (np.arange(64)[:, None] == _SRCY[None, :]).astype(np.float32)
_SA = (np.arange(32)[:, None] == _SRCA[None, :]).astype(np.float32)
_G16 = np.kron(np.eye(NH, dtype=np.float32), np.ones((16, 16), np.float32))


def _dot(a, b, prec=None):
    return jax.lax.dot_general(a, b, (((1,), (0,)), ((), ())),
                               precision=prec or _PREC,
                               preferred_element_type=jnp.float32)


def _val_body(q_ref, wv_ref, bv_ref, woff_ref, boff_ref, wat_ref, bat_ref,
              sx_ref, sy_ref, sa_ref,
              val_ref, wox_ref, box_ref, woy_ref, boy_ref,
              watw_ref, batw_ref):
    val_ref[...] = _dot(q_ref[...], wv_ref[...]) + bv_ref[...]
    sx, sy, sa = sx_ref[...], sy_ref[...], sa_ref[...]
    wox_ref[...] = _dot(woff_ref[...], sx, _EXACT)
    woy_ref[...] = _dot(woff_ref[...], sy, _EXACT)
    box_ref[...] = _dot(boff_ref[...], sx, _EXACT)
    boy_ref[...] = _dot(boff_ref[...], sy, _EXACT)
    watw_ref[...] = _dot(wat_ref[...], sa, _EXACT)
    batw_ref[...] = _dot(bat_ref[...], sa, _EXACT)


_val_call = pl.pallas_call(
    _val_body,
    grid=(NQ // VBLK,),
    in_specs=[
        pl.BlockSpec((VBLK, E), lambda i: (i, 0)),
        pl.BlockSpec((E, E), lambda i: (0, 0)),
        pl.BlockSpec((1, E), lambda i: (0, 0)),
        pl.BlockSpec((E, 64), lambda i: (0, 0)),
        pl.BlockSpec((1, 64), lambda i: (0, 0)),
        pl.BlockSpec((E, 32), lambda i: (0, 0)),
        pl.BlockSpec((1, 32), lambda i: (0, 0)),
        pl.BlockSpec((64, E), lambda i: (0, 0)),
        pl.BlockSpec((64, E), lambda i: (0, 0)),
        pl.BlockSpec((32, E), lambda i: (0, 0)),
    ],
    out_specs=[
        pl.BlockSpec((VBLK, E), lambda i: (i, 0)),
        pl.BlockSpec((E, E), lambda i: (0, 0)),
        pl.BlockSpec((1, E), lambda i: (0, 0)),
        pl.BlockSpec((E, E), lambda i: (0, 0)),
        pl.BlockSpec((1, E), lambda i: (0, 0)),
        pl.BlockSpec((E, E), lambda i: (0, 0)),
        pl.BlockSpec((1, E), lambda i: (0, 0)),
    ],
    out_shape=[
        jax.ShapeDtypeStruct((NQ, E), jnp.float32),
        jax.ShapeDtypeStruct((E, E), jnp.float32),
        jax.ShapeDtypeStruct((1, E), jnp.float32),
        jax.ShapeDtypeStruct((E, E), jnp.float32),
        jax.ShapeDtypeStruct((1, E), jnp.float32),
        jax.ShapeDtypeStruct((E, E), jnp.float32),
        jax.ShapeDtypeStruct((1, E), jnp.float32),
    ],
)


def _pre_body(qbase, q_ref, wox_ref, box_ref, woy_ref, boy_ref,
              wat_ref, bat_ref, g16_ref, idx_ref, wts_ref):
    i = pl.program_id(0)
    q = q_ref[...]
    ox = _dot(q, wox_ref[...]) + box_ref[...]
    oy = _dot(q, woy_ref[...]) + boy_ref[...]
    lg = _dot(q, wat_ref[...]) + bat_ref[...]
    m = jnp.max(lg, axis=1, keepdims=True)
    e = jnp.exp(lg - m)
    aw = 4.0 * e / _dot(e, g16_ref[...])

    lane = lax.broadcasted_iota(jnp.int32, (BLK, E), 1)
    xsel = ((lane % 16) // 4 % 2).astype(jnp.float32)
    ysel = ((lane % 16) // 8).astype(jnp.float32)
    hf = (lane // 16).astype(jnp.float32)
    qi = qbase + i * BLK + lax.broadcasted_iota(jnp.int32, (BLK, E), 0)
    refx = (qi % BEV).astype(jnp.float32) * (1.0 / 199.0)
    refy = (qi // BEV).astype(jnp.float32) * (1.0 / 199.0)

    x = refx * 200.0 - 0.5 + ox
    y = refy * 200.0 - 0.5 + oy
    x0 = jnp.floor(x)
    y0 = jnp.floor(y)
    fx = x - x0
    fy = y - y0
    xf = x0 + xsel
    yf = y0 + ysel
    wx = (1.0 - xsel) + (2.0 * xsel - 1.0) * fx
    wy = (1.0 - ysel) + (2.0 * ysel - 1.0) * fy
    v = ((xf >= 0.0) & (xf <= 199.0) & (yf >= 0.0) & (yf <= 199.0))
    xc = jnp.clip(xf, 0.0, 199.0)
    yc = jnp.clip(yf, 0.0, 199.0)

    idx_ref[...] = ((yc * 200.0 + xc) * 8.0 + hf).astype(jnp.int32)
    wts_ref[...] = aw * wx * wy * v.astype(jnp.float32)


def _make_pre(qbase, nqh):
    qb = qbase // BLK
    return pl.pallas_call(
        functools.partial(_pre_body, qbase),
        grid=(nqh // BLK,),
        in_specs=[
            pl.BlockSpec((BLK, E), lambda i: (i + qb, 0)),
            pl.BlockSpec((E, E), lambda i: (0, 0)),
            pl.BlockSpec((1, E), lambda i: (0, 0)),
            pl.BlockSpec((E, E), lambda i: (0, 0)),
            pl.BlockSpec((1, E), lambda i: (0, 0)),
            pl.BlockSpec((E, E), lambda i: (0, 0)),
            pl.BlockSpec((1, E), lambda i: (0, 0)),
            pl.BlockSpec((E, E), lambda i: (0, 0)),
        ],
        out_specs=[
            pl.BlockSpec((BLK, E), lambda i: (i, 0)),
            pl.BlockSpec((BLK, E), lambda i: (i, 0)),
        ],
        out_shape=[
            jax.ShapeDtypeStruct((nqh, E), jnp.int32),
            jax.ShapeDtypeStruct((nqh, E), jnp.float32),
        ],
    )


def _post_body(s_ref, wo_ref, bo_ref, q_ref, out_ref):
    out_ref[...] = (_dot(s_ref[...], wo_ref[...]) + bo_ref[...]
                    + 2.0 * q_ref[...])


def _post_body_alias(s_ref, wo_ref, bo_ref, q_ref, prev_ref, out_ref):
    out_ref[...] = (_dot(s_ref[...], wo_ref[...]) + bo_ref[...]
                    + 2.0 * q_ref[...])


def _make_post(qbase, nqh, alias):
    qb = qbase // BLK
    in_specs = [
        pl.BlockSpec((BLK, E), lambda i: (i, 0)),
        pl.BlockSpec((E, E), lambda i: (0, 0)),
        pl.BlockSpec((1, E), lambda i: (0, 0)),
        pl.BlockSpec((BLK, E), lambda i: (i + qb, 0)),
    ]
    kwargs = {}
    body = _post_body
    if alias:
        body = _post_body_alias
        in_specs.append(pl.BlockSpec((8, E), lambda i: (0, 0)))
        kwargs["input_output_aliases"] = {4: 0}
    return pl.pallas_call(
        body,
        grid=(nqh // BLK,),
        in_specs=in_specs,
        out_specs=pl.BlockSpec((BLK, E), lambda i: (i + qb, 0)),
        out_shape=jax.ShapeDtypeStruct((NQ, E), jnp.float32),
        **kwargs,
    )


def _make_sc_body(npairh):
    pw = npairh // NW
    nch = pw // KP
    tile_idxrows = pw * 16 // 128

    def _sc_body(tab_ref, idx_ref, w_ref, out_ref,
                 idx_v, w_v, rows_v, out_v, sidx, sgat, sout):
        wid = lax.axis_index("s") * NC + lax.axis_index("c")
        pairbase = wid * pw
        idxbase = wid * tile_idxrows

        def fire_idx(g):
            r = lax.rem(g, 3)
            pltpu.async_copy(idx_ref.at[pl.ds(idxbase + g * NJ, NJ)],
                             idx_v.at[pl.ds(r * NJ, NJ)], sidx)
            pltpu.async_copy(w_ref.at[pl.ds(pairbase + g * KP, KP)],
                             w_v.at[pl.ds(r * KP, KP)], sidx)

        def drain_idx():
            pltpu.make_async_copy(idx_ref.at[pl.ds(0, NJ)],
                                  idx_v.at[pl.ds(0, NJ)], sidx).wait()
            pltpu.make_async_copy(w_ref.at[pl.ds(0, KP)],
                                  w_v.at[pl.ds(0, KP)], sidx).wait()

        def fire_gat(g, b):
            r = lax.rem(g, 3)
            for j in range(NJ):
                pltpu.async_copy(tab_ref.at[idx_v.at[r * NJ + j]],
                                 rows_v.at[pl.ds(b * KP * 16 + j * 128, 128)],
                                 sgat)

        def drain_gat():
            pltpu.make_async_copy(tab_ref.at[pl.ds(0, KP * 16)],
                                  rows_v.at[pl.ds(0, KP * 16)], sgat).wait()

        def drain_out():
            pltpu.make_async_copy(out_v.at[pl.ds(0, KP)],
                                  out_ref.at[pl.ds(0, KP)], sout).wait()

        fire_idx(0)
        drain_idx()
        fire_gat(0, 0)
        fire_idx(1)

        def loop(g, carry):
            b = lax.rem(g, 2)

            @pl.when(g + 1 < nch)
            def _():
                drain_idx()
                fire_gat(g + 1, 1 - b)

            @pl.when(g + 2 < nch)
            def _():
                fire_idx(g + 2)

            @pl.when(g >= 2)
            def _():
                drain_out()

            drain_gat()

            rbase = b * KP * 16
            wbase = lax.rem(g, 3) * KP
            obase = b * KP

            def pair2(i2, c2):
                i = 2 * i2
                base = rbase + i * 16
                wv0 = w_v[wbase + i]
                wv1 = w_v[wbase + i + 1]
                a0 = rows_v[base + 0] * wv0[0]
                a1 = rows_v[base + 1] * wv0[1]
                a2 = rows_v[base + 2] * wv0[2]
                a3 = rows_v[base + 3] * wv0[3]
                b0 = rows_v[base + 16] * wv1[0]
                b1 = rows_v[base + 17] * wv1[1]
                b2 = rows_v[base + 18] * wv1[2]
                b3 = rows_v[base + 19] * wv1[3]
                for l in range(4, 16, 4):
                    a0 = a0 + rows_v[base + l] * wv0[l]
                    a1 = a1 + rows_v[base + l + 1] * wv0[l + 1]
                    a2 = a2 + rows_v[base + l + 2] * wv0[l + 2]
                    a3 = a3 + rows_v[base + l + 3] * wv0[l + 3]
                    b0 = b0 + rows_v[base + 16 + l] * wv1[l]
                    b1 = b1 + rows_v[base + 17 + l] * wv1[l + 1]
                    b2 = b2 + rows_v[base + 18 + l] * wv1[l + 2]
                    b3 = b3 + rows_v[base + 19 + l] * wv1[l + 3]
                out_v[obase + i] = (a0 + a1) + (a2 + a3)
                out_v[obase + i + 1] = (b0 + b1) + (b2 + b3)
                return c2

            lax.fori_loop(0, KP // 2, pair2, 0)
            pltpu.async_copy(out_v.at[pl.ds(obase, KP)],
                             out_ref.at[pl.ds(pairbase + g * KP, KP)], sout)
            return carry

        lax.fori_loop(0, nch, loop, 0)
        drain_out()
        drain_out()

    return _sc_body


@functools.cache
def _sc_call(npairh):
    return functools.partial(
        pl.kernel,
        out_type=jax.ShapeDtypeStruct((npairh, HD), jnp.float32),
        mesh=plsc.VectorSubcoreMesh(core_axis_name="c", subcore_axis_name="s"),
        compiler_params=pltpu.CompilerParams(use_tc_tiling_on_sc=False),
        scratch_types=[
            pltpu.VMEM((3 * NJ, 128), jnp.int32),
            pltpu.VMEM((3 * KP, HD), jnp.float32),
            pltpu.VMEM((2 * KP * 16, HD), jnp.float32),
            pltpu.VMEM((2 * KP, HD), jnp.float32),
            pltpu.SemaphoreType.DMA,
            pltpu.SemaphoreType.DMA,
            pltpu.SemaphoreType.DMA,
        ],
    )(_make_sc_body(npairh))


_pre_calls = []
_post_calls = []
_qb = 0
for _h, _nqh in enumerate(SPLITS):
    _pre_calls.append(_make_pre(_qb, _nqh))
    _post_calls.append(_make_post(_qb, _nqh, alias=_h > 0))
    _qb += _nqh


def kernel(query, Wv, bv, Woff, boff, Wattn, battn, Wout, bout):
    q2 = query[0]
    g16 = jnp.asarray(_G16)
    sx, sy, sa = jnp.asarray(_SX), jnp.asarray(_SY), jnp.asarray(_SA)

    value, wox, box, woy, boy, wat, bat = _val_call(
        q2, Wv, bv[None, :], Woff, boff[None, :], Wattn, battn[None, :],
        sx, sy, sa)
    tab = value.reshape(NPAIR, HD)

    out = None
    for h, nqh in enumerate(SPLITS):
        widx, wwts = _pre_calls[h](q2, wox, box, woy, boy, wat, bat, g16)
        srows = _sc_call(nqh * NH)(tab, widx, wwts.reshape(nqh * NH, HD))
        args = (srows.reshape(nqh, E), Wout, bout[None, :], q2)
        out = _post_calls[h](*args) if h == 0 else _post_calls[h](*args, out)
    return out[None]


# speedup vs baseline: 1.1210x; 1.0055x over previous
import functools

import numpy as np
import jax
import jax.numpy as jnp
from jax import lax
from jax.experimental import pallas as pl
from jax.experimental.pallas import tpu as pltpu
from jax.experimental.pallas import tpu_sc as plsc

BEV = 200
NQ = BEV * BEV
E = 128
NH = 8
HD = 16
NPAIR = NQ * NH

BLK = 1000
VBLK = 2000
SPLITS = (8000, 32000)

NC, NS = 2, 16
NW = NC * NS
KP = 200
NJ = KP * 16 // 128

_PREC = jax.lax.Precision.DEFAULT
_EXACT = jax.lax.Precision.HIGHEST

_LANE = np.arange(128)
_LH = _LANE // 16
_LP = _LANE % 4
_SRCX = _LH * 8 + _LP * 2 + 0
_SRCY = _LH * 8 + _LP * 2 + 1
_SRCA = _LH * 4 + _LP
_SX = (np.arange(64)[:, None] == _SRCX[None, :]).astype(np.float32)
_SY = (np.arange(64)[:, None] == _SRCY[None, :]).astype(np.float32)
_SA = (np.arange(32)[:, None] == _SRCA[None, :]).astype(np.float32)
_G16 = np.kron(np.eye(NH, dtype=np.float32), np.ones((16, 16), np.float32))


def _dot(a, b, prec=None):
    return jax.lax.dot_general(a, b, (((1,), (0,)), ((), ())),
                               precision=prec or _PREC,
                               preferred_element_type=jnp.float32)


def _val_body(q_ref, wv_ref, bv_ref, woff_ref, boff_ref, wat_ref, bat_ref,
              sx_ref, sy_ref, sa_ref,
              val_ref, wox_ref, box_ref, woy_ref, boy_ref,
              watw_ref, batw_ref):
    val_ref[...] = _dot(q_ref[...], wv_ref[...]) + bv_ref[...]
    sx, sy, sa = sx_ref[...], sy_ref[...], sa_ref[...]
    wox_ref[...] = _dot(woff_ref[...], sx, _EXACT)
    woy_ref[...] = _dot(woff_ref[...], sy, _EXACT)
    box_ref[...] = _dot(boff_ref[...], sx, _EXACT)
    boy_ref[...] = _dot(boff_ref[...], sy, _EXACT)
    watw_ref[...] = _dot(wat_ref[...], sa, _EXACT)
    batw_ref[...] = _dot(bat_ref[...], sa, _EXACT)


_val_call = pl.pallas_call(
    _val_body,
    grid=(NQ // VBLK,),
    in_specs=[
        pl.BlockSpec((VBLK, E), lambda i: (i, 0)),
        pl.BlockSpec((E, E), lambda i: (0, 0)),
        pl.BlockSpec((1, E), lambda i: (0, 0)),
        pl.BlockSpec((E, 64), lambda i: (0, 0)),
        pl.BlockSpec((1, 64), lambda i: (0, 0)),
        pl.BlockSpec((E, 32), lambda i: (0, 0)),
        pl.BlockSpec((1, 32), lambda i: (0, 0)),
        pl.BlockSpec((64, E), lambda i: (0, 0)),
        pl.BlockSpec((64, E), lambda i: (0, 0)),
        pl.BlockSpec((32, E), lambda i: (0, 0)),
    ],
    out_specs=[
        pl.BlockSpec((VBLK, E), lambda i: (i, 0)),
        pl.BlockSpec((E, E), lambda i: (0, 0)),
        pl.BlockSpec((1, E), lambda i: (0, 0)),
        pl.BlockSpec((E, E), lambda i: (0, 0)),
        pl.BlockSpec((1, E), lambda i: (0, 0)),
        pl.BlockSpec((E, E), lambda i: (0, 0)),
        pl.BlockSpec((1, E), lambda i: (0, 0)),
    ],
    out_shape=[
        jax.ShapeDtypeStruct((NQ, E), jnp.float32),
        jax.ShapeDtypeStruct((E, E), jnp.float32),
        jax.ShapeDtypeStruct((1, E), jnp.float32),
        jax.ShapeDtypeStruct((E, E), jnp.float32),
        jax.ShapeDtypeStruct((1, E), jnp.float32),
        jax.ShapeDtypeStruct((E, E), jnp.float32),
        jax.ShapeDtypeStruct((1, E), jnp.float32),
    ],
)


def _pre_body(qbase, q_ref, wox_ref, box_ref, woy_ref, boy_ref,
              wat_ref, bat_ref, g16_ref, idx_ref, wts_ref):
    i = pl.program_id(0)
    q = q_ref[...]
    ox = _dot(q, wox_ref[...]) + box_ref[...]
    oy = _dot(q, woy_ref[...]) + boy_ref[...]
    lg = _dot(q, wat_ref[...]) + bat_ref[...]
    m = jnp.max(lg, axis=1, keepdims=True)
    e = jnp.exp(lg - m)
    aw = 4.0 * e / _dot(e, g16_ref[...])

    lane = lax.broadcasted_iota(jnp.int32, (BLK, E), 1)
    xsel = ((lane % 16) // 4 % 2).astype(jnp.float32)
    ysel = ((lane % 16) // 8).astype(jnp.float32)
    hf = (lane // 16).astype(jnp.float32)
    qi = qbase + i * BLK + lax.broadcasted_iota(jnp.int32, (BLK, E), 0)
    refx = (qi % BEV).astype(jnp.float32) * (1.0 / 199.0)
    refy = (qi // BEV).astype(jnp.float32) * (1.0 / 199.0)

    x = refx * 200.0 - 0.5 + ox
    y = refy * 200.0 - 0.5 + oy
    x0 = jnp.floor(x)
    y0 = jnp.floor(y)
    fx = x - x0
    fy = y - y0
    xf = x0 + xsel
    yf = y0 + ysel
    wx = (1.0 - xsel) + (2.0 * xsel - 1.0) * fx
    wy = (1.0 - ysel) + (2.0 * ysel - 1.0) * fy
    v = ((xf >= 0.0) & (xf <= 199.0) & (yf >= 0.0) & (yf <= 199.0))
    xc = jnp.clip(xf, 0.0, 199.0)
    yc = jnp.clip(yf, 0.0, 199.0)

    idx_ref[...] = ((yc * 200.0 + xc) * 8.0 + hf).astype(jnp.int32)
    wts_ref[...] = aw * wx * wy * v.astype(jnp.float32)


def _make_pre(qbase, nqh):
    qb = qbase // BLK
    return pl.pallas_call(
        functools.partial(_pre_body, qbase),
        grid=(nqh // BLK,),
        in_specs=[
            pl.BlockSpec((BLK, E), lambda i: (i + qb, 0)),
            pl.BlockSpec((E, E), lambda i: (0, 0)),
            pl.BlockSpec((1, E), lambda i: (0, 0)),
            pl.BlockSpec((E, E), lambda i: (0, 0)),
            pl.BlockSpec((1, E), lambda i: (0, 0)),
            pl.BlockSpec((E, E), lambda i: (0, 0)),
            pl.BlockSpec((1, E), lambda i: (0, 0)),
            pl.BlockSpec((E, E), lambda i: (0, 0)),
        ],
        out_specs=[
            pl.BlockSpec((BLK, E), lambda i: (i, 0)),
            pl.BlockSpec((BLK, E), lambda i: (i, 0)),
        ],
        out_shape=[
            jax.ShapeDtypeStruct((nqh, E), jnp.int32),
            jax.ShapeDtypeStruct((nqh, E), jnp.float32),
        ],
    )


def _post_body(s_ref, wo_ref, bo_ref, q_ref, out_ref):
    out_ref[...] = (_dot(s_ref[...], wo_ref[...]) + bo_ref[...]
                    + 2.0 * q_ref[...])


def _post_body_alias(s_ref, wo_ref, bo_ref, q_ref, prev_ref, out_ref):
    out_ref[...] = (_dot(s_ref[...], wo_ref[...]) + bo_ref[...]
                    + 2.0 * q_ref[...])


def _make_post(qbase, nqh, alias):
    qb = qbase // BLK
    in_specs = [
        pl.BlockSpec((BLK, E), lambda i: (i, 0)),
        pl.BlockSpec((E, E), lambda i: (0, 0)),
        pl.BlockSpec((1, E), lambda i: (0, 0)),
        pl.BlockSpec((BLK, E), lambda i: (i + qb, 0)),
    ]
    kwargs = {}
    body = _post_body
    if alias:
        body = _post_body_alias
        in_specs.append(pl.BlockSpec((8, E), lambda i: (0, 0)))
        kwargs["input_output_aliases"] = {4: 0}
    return pl.pallas_call(
        body,
        grid=(nqh // BLK,),
        in_specs=in_specs,
        out_specs=pl.BlockSpec((BLK, E), lambda i: (i + qb, 0)),
        out_shape=jax.ShapeDtypeStruct((NQ, E), jnp.float32),
        **kwargs,
    )


def _make_sc_body(npairh):
    pw = npairh // NW
    nch = pw // KP
    tile_idxrows = pw * 16 // 128

    def _sc_body(tab_ref, idx_ref, w_ref, out_ref,
                 idx_v, w_v, rows_v, out_v, sidx, sgat, sout):
        wid = lax.axis_index("s") * NC + lax.axis_index("c")
        pairbase = wid * pw
        idxbase = wid * tile_idxrows

        def fire_idx(g):
            r = lax.rem(g, 3)
            pltpu.async_copy(idx_ref.at[pl.ds(idxbase + g * NJ, NJ)],
                             idx_v.at[pl.ds(r * NJ, NJ)], sidx)
            pltpu.async_copy(w_ref.at[pl.ds(pairbase + g * KP, KP)],
                             w_v.at[pl.ds(r * KP, KP)], sidx)

        def drain_idx():
            pltpu.make_async_copy(idx_ref.at[pl.ds(0, NJ)],
                                  idx_v.at[pl.ds(0, NJ)], sidx).wait()
            pltpu.make_async_copy(w_ref.at[pl.ds(0, KP)],
                                  w_v.at[pl.ds(0, KP)], sidx).wait()

        def fire_gat(g, b):
            r = lax.rem(g, 3)
            for j in range(NJ):
                pltpu.async_copy(tab_ref.at[idx_v.at[r * NJ + j]],
                                 rows_v.at[pl.ds(b * KP * 16 + j * 128, 128)],
                                 sgat)

        def drain_gat():
            pltpu.make_async_copy(tab_ref.at[pl.ds(0, KP * 16)],
                                  rows_v.at[pl.ds(0, KP * 16)], sgat).wait()

        def drain_out():
            pltpu.make_async_copy(out_v.at[pl.ds(0, KP)],
                                  out_ref.at[pl.ds(0, KP)], sout).wait()

        fire_idx(0)
        drain_idx()
        fire_gat(0, 0)
        fire_idx(1)

        def loop(g, carry):
            b = lax.rem(g, 2)

            @pl.when(g + 1 < nch)
            def _():
                drain_idx()
                fire_gat(g + 1, 1 - b)

            @pl.when(g + 2 < nch)
            def _():
                fire_idx(g + 2)

            @pl.when(g >= 2)
            def _():
                drain_out()

            drain_gat()

            rbase = b * KP * 16
            wbase = lax.rem(g, 3) * KP
            obase = b * KP

            def pair2(i2, c2):
                i = 2 * i2
                base = rbase + i * 16
                wv0 = w_v[wbase + i]
                wv1 = w_v[wbase + i + 1]
                a0 = rows_v[base + 0] * wv0[0]
                a1 = rows_v[base + 1] * wv0[1]
                a2 = rows_v[base + 2] * wv0[2]
                a3 = rows_v[base + 3] * wv0[3]
                b0 = rows_v[base + 16] * wv1[0]
                b1 = rows_v[base + 17] * wv1[1]
                b2 = rows_v[base + 18] * wv1[2]
                b3 = rows_v[base + 19] * wv1[3]
                for l in range(4, 16, 4):
                    a0 = a0 + rows_v[base + l] * wv0[l]
                    a1 = a1 + rows_v[base + l + 1] * wv0[l + 1]
                    a2 = a2 + rows_v[base + l + 2] * wv0[l + 2]
                    a3 = a3 + rows_v[base + l + 3] * wv0[l + 3]
                    b0 = b0 + rows_v[base + 16 + l] * wv1[l]
                    b1 = b1 + rows_v[base + 17 + l] * wv1[l + 1]
                    b2 = b2 + rows_v[base + 18 + l] * wv1[l + 2]
                    b3 = b3 + rows_v[base + 19 + l] * wv1[l + 3]
                out_v[obase + i] = (a0 + a1) + (a2 + a3)
                out_v[obase + i + 1] = (b0 + b1) + (b2 + b3)
                return c2

            lax.fori_loop(0, KP // 2, pair2, 0)
            pltpu.async_copy(out_v.at[pl.ds(obase, KP)],
                             out_ref.at[pl.ds(pairbase + g * KP, KP)], sout)
            return carry

        lax.fori_loop(0, nch, loop, 0)
        drain_out()
        drain_out()

    return _sc_body


@functools.cache
def _sc_call(npairh):
    return functools.partial(
        pl.kernel,
        out_type=jax.ShapeDtypeStruct((npairh, HD), jnp.float32),
        mesh=plsc.VectorSubcoreMesh(core_axis_name="c", subcore_axis_name="s"),
        compiler_params=pltpu.CompilerParams(
            use_tc_tiling_on_sc=False,
            disable_bounds_checks=True,
            disable_semaphore_checks=True,
        ),
        scratch_types=[
            pltpu.VMEM((3 * NJ, 128), jnp.int32),
            pltpu.VMEM((3 * KP, HD), jnp.float32),
            pltpu.VMEM((2 * KP * 16, HD), jnp.float32),
            pltpu.VMEM((2 * KP, HD), jnp.float32),
            pltpu.SemaphoreType.DMA,
            pltpu.SemaphoreType.DMA,
            pltpu.SemaphoreType.DMA,
        ],
    )(_make_sc_body(npairh))


_pre_calls = []
_post_calls = []
_qb = 0
for _h, _nqh in enumerate(SPLITS):
    _pre_calls.append(_make_pre(_qb, _nqh))
    _post_calls.append(_make_post(_qb, _nqh, alias=_h > 0))
    _qb += _nqh


def kernel(query, Wv, bv, Woff, boff, Wattn, battn, Wout, bout):
    q2 = query[0]
    g16 = jnp.asarray(_G16)
    sx, sy, sa = jnp.asarray(_SX), jnp.asarray(_SY), jnp.asarray(_SA)

    value, wox, box, woy, boy, wat, bat = _val_call(
        q2, Wv, bv[None, :], Woff, boff[None, :], Wattn, battn[None, :],
        sx, sy, sa)
    tab = value.reshape(NPAIR, HD)

    out = None
    for h, nqh in enumerate(SPLITS):
        widx, wwts = _pre_calls[h](q2, wox, box, woy, boy, wat, bat, g16)
        srows = _sc_call(nqh * NH)(tab, widx, wwts.reshape(nqh * NH, HD))
        args = (srows.reshape(nqh, E), Wout, bout[None, :], q2)
        out = _post_calls[h](*args) if h == 0 else _post_calls[h](*args, out)
    return out[None]
